# baseline (speedup 1.0000x reference)
import jax
import jax.numpy as jnp
from jax import lax
from jax.experimental import pallas as pl

NC = 1000
ROWS = 1024 * 26
BLK = 512


def _onehot_block(idx_ref, out_ref):
    idx = idx_ref[0, 0, :]
    iota = lax.broadcasted_iota(jnp.int32, (BLK, NC), 1)
    out_ref[...] = (idx[:, None] == iota).astype(out_ref.dtype)


def kernel(x):
    dt = x.dtype
    xf = x.reshape(ROWS // BLK, 1, BLK)
    out = pl.pallas_call(
        _onehot_block,
        grid=(ROWS // BLK,),
        in_specs=[pl.BlockSpec((1, 1, BLK), lambda i: (i, 0, 0))],
        out_specs=pl.BlockSpec((BLK, NC), lambda i: (i, 0)),
        out_shape=jax.ShapeDtypeStruct((ROWS, NC), dt),
    )(xf)
    return out.reshape(1024, 26, NC)


# speedup vs baseline: 1.4497x; 1.4497x over previous
import jax
import jax.numpy as jnp
from jax import lax
from jax.experimental import pallas as pl

NC = 1000
B0 = 1024
B1 = 26
R = 128


def _onehot_block(idx_ref, out_ref):
    idx = idx_ref[...]
    iota = lax.broadcasted_iota(jnp.int32, (R, B1, NC), 2)
    out_ref[...] = (idx[:, :, None] == iota).astype(out_ref.dtype)


def kernel(x):
    return pl.pallas_call(
        _onehot_block,
        grid=(B0 // R,),
        in_specs=[pl.BlockSpec((R, B1), lambda i: (i, 0))],
        out_specs=pl.BlockSpec((R, B1, NC), lambda i: (i, 0, 0)),
        out_shape=jax.ShapeDtypeStruct((B0, B1, NC), x.dtype),
    )(x)


# speedup vs baseline: 1.4534x; 1.0025x over previous
import jax
import jax.numpy as jnp
from jax import lax
from jax.experimental import pallas as pl
from jax.experimental.pallas import tpu as pltpu

NC = 1000
B0 = 1024
B1 = 26
RB = 32
NSTEP = B0 // RB
NBUF = 8


def _onehot_body(idx_ref, out_ref, scratch, sems):
    i = pl.program_id(0)
    slot = lax.rem(i, NBUF)

    @pl.when(i >= NBUF)
    def _wait_prev():
        pltpu.make_async_copy(
            scratch.at[slot],
            out_ref.at[pl.ds((i - NBUF) * RB, RB)],
            sems.at[slot],
        ).wait()

    idx = idx_ref[pl.ds(i * RB, RB), :]
    iota = lax.broadcasted_iota(jnp.int32, (RB, B1, NC), 2)
    scratch[slot] = (idx[:, :, None] == iota).astype(scratch.dtype)

    pltpu.make_async_copy(
        scratch.at[slot],
        out_ref.at[pl.ds(i * RB, RB)],
        sems.at[slot],
    ).start()

    @pl.when(i == NSTEP - 1)
    def _drain():
        for k in range(NBUF):
            step = NSTEP - NBUF + k
            pltpu.make_async_copy(
                scratch.at[k],
                out_ref.at[pl.ds(step * RB, RB)],
                sems.at[k],
            ).wait()


def kernel(x):
    return pl.pallas_call(
        _onehot_body,
        grid=(NSTEP,),
        in_specs=[pl.BlockSpec((B0, B1), lambda i: (0, 0))],
        out_specs=pl.BlockSpec(memory_space=pltpu.MemorySpace.HBM),
        out_shape=jax.ShapeDtypeStruct((B0, B1, NC), x.dtype),
        scratch_shapes=[
            pltpu.MemorySpace.VMEM((NBUF, RB, B1, NC), jnp.int32),
            pltpu.SemaphoreType.DMA((NBUF,)),
        ],
        compiler_params=pltpu.CompilerParams(
            vmem_limit_bytes=100 * 1024 * 1024,
        ),
    )(x)


# speedup vs baseline: 7.2368x; 4.9793x over previous
import jax
import jax.numpy as jnp
from jax import lax
from jax.experimental import pallas as pl

NC = 1000
B0 = 1024
B1 = 26


def _onehot_t(idxt_ref, out_ref):
    i = pl.program_id(0)
    row = idxt_ref[pl.ds(i, 1), :]
    iota = lax.broadcasted_iota(jnp.int32, (1, NC, B0), 1)
    cmp = row[:, None, :] == iota
    out_ref[...] = cmp.astype(out_ref.dtype)


def kernel(x):
    xt = x.T
    out_t = pl.pallas_call(
        _onehot_t,
        grid=(B1,),
        in_specs=[pl.BlockSpec((B1, B0), lambda i: (0, 0))],
        out_specs=pl.BlockSpec((1, NC, B0), lambda i: (i, 0, 0)),
        out_shape=jax.ShapeDtypeStruct((B1, NC, B0), x.dtype),
    )(xt)
    return jnp.transpose(out_t, (2, 0, 1))
